# Initial kernel scaffold; baseline (speedup 1.0000x reference)
import jax
import jax.numpy as jnp
from jax.experimental import pallas as pl


def kernel(x, edge_index, fold_n, layer, edge_attr, Wq, bq, Wk, bk, Wv, bv, We, Wskip, bskip):
    raise NotImplementedError("write your pallas kernel here")


# speedup vs baseline: 6.3958x; 6.3958x over previous
import functools

import jax
import jax.numpy as jnp
from jax import lax
from jax.experimental import pallas as pl
from jax.experimental.pallas import tpu as pltpu
from jax.experimental.pallas import tpu_sc as plsc

N = 10000
E = 320000
D = 128
H = 4
C = 32
HC = H * C
ED = 16
ACCW = 144
INV_SQRT_C = 1.0 / (C ** 0.5)

NC = 2
NS = 16
NW = NC * NS
EPW = E // NW
CH = 48
NCHUNK = EPW // CH
TAIL = EPW - NCHUNK * CH
GRP = CH // 16
RPT = N // NS


def _proj_body(x_ref, wq_ref, bq_ref, wk_ref, bk_ref, wv_ref, bv_ref,
               q_ref, k_ref, v_ref):
    x = x_ref[...]
    q_ref[...] = jnp.dot(x, wq_ref[...], preferred_element_type=jnp.float32) + bq_ref[...]
    k_ref[...] = jnp.dot(x, wk_ref[...], preferred_element_type=jnp.float32) + bk_ref[...]
    v_ref[...] = jnp.dot(x, wv_ref[...], preferred_element_type=jnp.float32) + bv_ref[...]


def _projections(x, Wq, bq, Wk, bk, Wv, bv):
    bn = 2000
    grid = (N // bn,)
    row_spec = pl.BlockSpec((bn, D), lambda i: (i, 0))
    w_spec = pl.BlockSpec((D, HC), lambda i: (0, 0))
    b_spec = pl.BlockSpec((1, HC), lambda i: (0, 0))
    return pl.pallas_call(
        _proj_body,
        grid=grid,
        in_specs=[row_spec, w_spec, b_spec, w_spec, b_spec, w_spec, b_spec],
        out_specs=[row_spec, row_spec, row_spec],
        out_shape=[jax.ShapeDtypeStruct((N, HC), jnp.float32)] * 3,
    )(x, Wq, bq.reshape(1, HC), Wk, bk.reshape(1, HC), Wv, bv.reshape(1, HC))


def _eproj_body(ea_ref, we_ref, e_ref):
    e_ref[...] = jnp.dot(ea_ref[...], we_ref[...],
                         preferred_element_type=jnp.float32)


def _eproj(edge_attr, We):
    bn = 8000
    return pl.pallas_call(
        _eproj_body,
        grid=(E // bn,),
        in_specs=[pl.BlockSpec((bn, ED), lambda i: (i, 0)),
                  pl.BlockSpec((ED, HC), lambda i: (0, 0))],
        out_specs=pl.BlockSpec((bn, HC), lambda i: (i, 0)),
        out_shape=jax.ShapeDtypeStruct((E, HC), jnp.float32),
    )(edge_attr, We)


def _edge_body(src_hbm, dst_hbm, q_hbm, k_hbm, v_hbm, e_hbm, out_hbm,
               srcv, dstv, srcv2, dstv2, qb, kb, vb, eb, msg, acc, sem):
    cid = lax.axis_index("c")
    sid = lax.axis_index("s")
    wid = sid * NC + cid

    zero16 = jnp.zeros((16,), jnp.float32)
    lanes = lax.iota(jnp.int32, 16)

    def _mrow(r, carry):
        for cc in range(ACCW // 16):
            msg[r, pl.ds(cc * 16, 16)] = zero16
        return carry

    lax.fori_loop(0, CH, _mrow, 0)
    for i in range(25):
        pltpu.sync_copy(msg.at[pl.ds(0, 25)],
                        acc.at[pl.ds(sid * RPT + i * 25, 25)])
    plsc.subcore_barrier()

    def _group(g, rows_v, q_ref, k_ref, v_ref, e_ref):
        rows = g * 16 + lanes
        exs = []
        for h in range(H):
            a = jnp.zeros((16,), jnp.float32)
            for cc in range(C):
                col = jnp.full((16,), h * C + cc, jnp.int32)
                qv = plsc.load_gather(q_ref, [rows, col])
                kv = plsc.load_gather(k_ref, [rows, col])
                ev = plsc.load_gather(e_ref, [rows, col])
                a = a + qv * (kv + ev)
            ex = jnp.exp(a * INV_SQRT_C)
            exs.append(ex)
            plsc.store_scatter(
                msg, [rows, jnp.full((16,), HC + h, jnp.int32)], ex)
        for h in range(H):
            for cc in range(C):
                col = jnp.full((16,), h * C + cc, jnp.int32)
                vv = plsc.load_gather(v_ref, [rows, col])
                ev = plsc.load_gather(e_ref, [rows, col])
                plsc.store_scatter(msg, [rows, col], exs[h] * (vv + ev))

    def _chunk(t, carry):
        base = wid * EPW + t * CH
        pltpu.sync_copy(src_hbm.at[pl.ds(base, CH)], srcv)
        pltpu.sync_copy(dst_hbm.at[pl.ds(base, CH)], dstv)
        cps = [pltpu.async_copy(q_hbm.at[dstv], qb, sem),
               pltpu.async_copy(k_hbm.at[srcv], kb, sem),
               pltpu.async_copy(v_hbm.at[srcv], vb, sem),
               pltpu.async_copy(e_hbm.at[pl.ds(base, CH)], eb, sem)]
        for cp in cps:
            cp.wait()

        def _grp(g, gcarry):
            _group(g, None, qb, kb, vb, eb)
            return gcarry

        lax.fori_loop(0, GRP, _grp, 0)
        pltpu.sync_copy(msg, acc.at[dstv], add=True)
        return carry

    lax.fori_loop(0, NCHUNK, _chunk, 0)

    tbase = wid * EPW + NCHUNK * CH
    pltpu.sync_copy(src_hbm.at[pl.ds(tbase, TAIL)], srcv2)
    pltpu.sync_copy(dst_hbm.at[pl.ds(tbase, TAIL)], dstv2)
    cps = [pltpu.async_copy(q_hbm.at[dstv2], qb.at[pl.ds(0, TAIL)], sem),
           pltpu.async_copy(k_hbm.at[srcv2], kb.at[pl.ds(0, TAIL)], sem),
           pltpu.async_copy(v_hbm.at[srcv2], vb.at[pl.ds(0, TAIL)], sem),
           pltpu.async_copy(e_hbm.at[pl.ds(tbase, TAIL)],
                            eb.at[pl.ds(0, TAIL)], sem)]
    for cp in cps:
        cp.wait()
    _group(0, None, qb, kb, vb, eb)
    pltpu.sync_copy(msg.at[pl.ds(0, TAIL)], acc.at[dstv2], add=True)

    plsc.subcore_barrier()
    for i in range(5):
        off = sid * RPT + i * 125
        pltpu.sync_copy(acc.at[pl.ds(off, 125)], out_hbm.at[cid, pl.ds(off, 125)])


def _edge_phase(src, dst, q, k, v, e):
    mesh = plsc.VectorSubcoreMesh(core_axis_name="c", subcore_axis_name="s")
    fn = functools.partial(
        pl.kernel,
        out_type=jax.ShapeDtypeStruct((NC, N, ACCW), jnp.float32),
        mesh=mesh,
        scratch_types=[
            pltpu.VMEM((CH,), jnp.int32),
            pltpu.VMEM((CH,), jnp.int32),
            pltpu.VMEM((TAIL,), jnp.int32),
            pltpu.VMEM((TAIL,), jnp.int32),
            pltpu.VMEM((CH, HC), jnp.float32),
            pltpu.VMEM((CH, HC), jnp.float32),
            pltpu.VMEM((CH, HC), jnp.float32),
            pltpu.VMEM((CH, HC), jnp.float32),
            pltpu.VMEM((CH, ACCW), jnp.float32),
            pltpu.VMEM_SHARED((N, ACCW), jnp.float32),
            pltpu.SemaphoreType.DMA,
        ],
        compiler_params=pltpu.CompilerParams(use_tc_tiling_on_sc=False,
                                             needs_layout_passes=False),
    )(_edge_body)
    return fn(src, dst, q, k, v, e)


def _combine_body(p_ref, x_ref, ws_ref, bs_ref, o_ref):
    p = p_ref[...]
    s = p[0] + p[1]
    msg = s[:, :HC]
    den4 = s[:, HC:HC + H]
    lane_head = lax.broadcasted_iota(jnp.int32, (H, HC), 1) // C
    head_id = lax.broadcasted_iota(jnp.int32, (H, HC), 0)
    expand = (lane_head == head_id).astype(jnp.float32)
    den = jnp.dot(den4, expand, preferred_element_type=jnp.float32)
    o_ref[...] = (msg / (den + 1e-16)
                  + jnp.dot(x_ref[...], ws_ref[...],
                            preferred_element_type=jnp.float32)
                  + bs_ref[...])


def _combine(partials, x, Wskip, bskip):
    bn = 2000
    return pl.pallas_call(
        _combine_body,
        grid=(N // bn,),
        in_specs=[pl.BlockSpec((NC, bn, ACCW), lambda i: (0, i, 0)),
                  pl.BlockSpec((bn, D), lambda i: (i, 0)),
                  pl.BlockSpec((D, HC), lambda i: (0, 0)),
                  pl.BlockSpec((1, HC), lambda i: (0, 0))],
        out_specs=pl.BlockSpec((bn, HC), lambda i: (i, 0)),
        out_shape=jax.ShapeDtypeStruct((N, HC), jnp.float32),
    )(partials, x, Wskip, bskip.reshape(1, HC))


def kernel(x, edge_index, fold_n, layer, edge_attr,
           Wq, bq, Wk, bk, Wv, bv, We, Wskip, bskip):
    src = edge_index[0].astype(jnp.int32)
    dst = edge_index[1].astype(jnp.int32)
    q, k, v = _projections(x, Wq, bq, Wk, bk, Wv, bv)
    e = _eproj(edge_attr, We)
    partials = _edge_phase(src, dst, q, k, v, e)
    return _combine(partials, x, Wskip, bskip)


# speedup vs baseline: 7.1259x; 1.1142x over previous
import functools

import jax
import jax.numpy as jnp
from jax import lax
from jax.experimental import pallas as pl
from jax.experimental.pallas import tpu as pltpu
from jax.experimental.pallas import tpu_sc as plsc

N = 10000
E = 320000
D = 128
H = 4
C = 32
HC = H * C
KV = 2 * HC
ED = 16
ACCW = 144
INV_SQRT_C = 1.0 / (C ** 0.5)

NC = 2
NS = 16
NW = NC * NS
EPW = E // NW
CH = 32
NCHUNK = EPW // CH
TAIL = EPW - NCHUNK * CH
GRP = CH // 16
RPT = N // NS
NBUF = 2


def _proj_body(x_ref, wq_ref, bq_ref, wk_ref, bk_ref, wv_ref, bv_ref,
               q_ref, kv_ref):
    x = x_ref[...]
    q_ref[...] = jnp.dot(x, wq_ref[...], preferred_element_type=jnp.float32) + bq_ref[...]
    kv_ref[:, :HC] = jnp.dot(x, wk_ref[...], preferred_element_type=jnp.float32) + bk_ref[...]
    kv_ref[:, HC:] = jnp.dot(x, wv_ref[...], preferred_element_type=jnp.float32) + bv_ref[...]


def _projections(x, Wq, bq, Wk, bk, Wv, bv):
    bn = 2000
    grid = (N // bn,)
    row_spec = pl.BlockSpec((bn, D), lambda i: (i, 0))
    w_spec = pl.BlockSpec((D, HC), lambda i: (0, 0))
    b_spec = pl.BlockSpec((1, HC), lambda i: (0, 0))
    return pl.pallas_call(
        _proj_body,
        grid=grid,
        in_specs=[row_spec, w_spec, b_spec, w_spec, b_spec, w_spec, b_spec],
        out_specs=[row_spec, pl.BlockSpec((bn, KV), lambda i: (i, 0))],
        out_shape=[jax.ShapeDtypeStruct((N, HC), jnp.float32),
                   jax.ShapeDtypeStruct((N, KV), jnp.float32)],
    )(x, Wq, bq.reshape(1, HC), Wk, bk.reshape(1, HC), Wv, bv.reshape(1, HC))


def _eproj_body(ea_ref, we_ref, e_ref):
    e_ref[...] = jnp.dot(ea_ref[...], we_ref[...],
                         preferred_element_type=jnp.float32)


def _eproj(edge_attr, We):
    bn = 8000
    return pl.pallas_call(
        _eproj_body,
        grid=(E // bn,),
        in_specs=[pl.BlockSpec((bn, ED), lambda i: (i, 0)),
                  pl.BlockSpec((ED, HC), lambda i: (0, 0))],
        out_specs=pl.BlockSpec((bn, HC), lambda i: (i, 0)),
        out_shape=jax.ShapeDtypeStruct((E, HC), jnp.float32),
    )(edge_attr, We)


def _edge_body(ei_hbm, q_hbm, kv_hbm, e_hbm, out_hbm,
               idx0, idx1, qb0, qb1, kvb0, kvb1, eb0, eb1,
               srcv2, dstv2, msg, acc, gsem0, gsem1, sem):
    idxs = (idx0, idx1)
    qbs = (qb0, qb1)
    kvbs = (kvb0, kvb1)
    ebs = (eb0, eb1)
    gsems = (gsem0, gsem1)

    cid = lax.axis_index("c")
    sid = lax.axis_index("s")
    wid = sid * NC + cid
    ebase = wid * EPW

    zero16 = jnp.zeros((16,), jnp.float32)
    lanes = lax.iota(jnp.int32, 16)

    def _mrow(r, carry):
        for cc in range(ACCW // 16):
            msg[r, pl.ds(cc * 16, 16)] = zero16
        return carry

    lax.fori_loop(0, CH, _mrow, 0)
    for i in range(25):
        pltpu.sync_copy(msg.at[pl.ds(0, 25)],
                        acc.at[pl.ds(sid * RPT + i * 25, 25)])
    plsc.subcore_barrier()

    def _fire(b, t):
        base = ebase + t * CH
        pltpu.sync_copy(ei_hbm.at[:, pl.ds(base, CH)], idxs[b])
        pltpu.async_copy(q_hbm.at[idxs[b].at[1]], qbs[b], gsems[b])
        pltpu.async_copy(kv_hbm.at[idxs[b].at[0]], kvbs[b], gsems[b])
        pltpu.async_copy(e_hbm.at[pl.ds(base, CH)], ebs[b], gsems[b])

    def _drain(b, t):
        base = ebase + t * CH
        pltpu.make_async_copy(q_hbm.at[idxs[b].at[1]], qbs[b], gsems[b]).wait()
        pltpu.make_async_copy(kv_hbm.at[idxs[b].at[0]], kvbs[b], gsems[b]).wait()
        pltpu.make_async_copy(e_hbm.at[pl.ds(base, CH)], ebs[b], gsems[b]).wait()

    CU = 8

    def _group(g, q_ref, kv_ref, e_ref, nrow):
        for gg in range(nrow // 16):
            rows = g * 16 + gg * 16 + lanes
            exs = []
            for h in range(H):
                def _alpha(j, a):
                    base = j * CU + h * C
                    for k in range(CU):
                        col = jnp.full((16,), base + k, jnp.int32)
                        qv = plsc.load_gather(q_ref, [rows, col])
                        kvv = plsc.load_gather(kv_ref, [rows, col])
                        ev = plsc.load_gather(e_ref, [rows, col])
                        a = a + qv * (kvv + ev)
                    return a

                a = lax.fori_loop(0, C // CU, _alpha,
                                  jnp.zeros((16,), jnp.float32))
                ex = jnp.exp(a * INV_SQRT_C)
                exs.append(ex)
                plsc.store_scatter(
                    msg, [rows, jnp.full((16,), HC + h, jnp.int32)], ex)
            for h in range(H):
                ex = exs[h]

                def _msg(j, carry):
                    base = j * CU + h * C
                    for k in range(CU):
                        col = jnp.full((16,), base + k, jnp.int32)
                        colv = jnp.full((16,), base + k + HC, jnp.int32)
                        vv = plsc.load_gather(kv_ref, [rows, colv])
                        ev = plsc.load_gather(e_ref, [rows, col])
                        plsc.store_scatter(msg, [rows, col], ex * (vv + ev))
                    return carry

                lax.fori_loop(0, C // CU, _msg, 0)

    for b in range(NBUF):
        _fire(b, b)

    def _outer(i, carry):
        for b in range(NBUF):
            t = i * NBUF + b
            _drain(b, t)

            def _grp(g, gcarry):
                _group(g, qbs[b], kvbs[b], ebs[b], 16)
                return gcarry

            lax.fori_loop(0, GRP, _grp, 0)
            pltpu.sync_copy(msg, acc.at[idxs[b].at[1]], add=True)

            @pl.when(t + NBUF < NCHUNK)
            def _():
                _fire(b, t + NBUF)
        return carry

    lax.fori_loop(0, NCHUNK // NBUF, _outer, 0)

    tbase = ebase + NCHUNK * CH
    pltpu.sync_copy(ei_hbm.at[0, pl.ds(tbase, TAIL)], srcv2)
    pltpu.sync_copy(ei_hbm.at[1, pl.ds(tbase, TAIL)], dstv2)
    cps = [pltpu.async_copy(q_hbm.at[dstv2], qb0.at[pl.ds(0, TAIL)], sem),
           pltpu.async_copy(kv_hbm.at[srcv2], kvb0.at[pl.ds(0, TAIL)], sem),
           pltpu.async_copy(e_hbm.at[pl.ds(tbase, TAIL)],
                            eb0.at[pl.ds(0, TAIL)], sem)]
    for cp in cps:
        cp.wait()
    _group(0, qb0, kvb0, eb0, TAIL)
    pltpu.sync_copy(msg.at[pl.ds(0, TAIL)], acc.at[dstv2], add=True)

    plsc.subcore_barrier()
    for i in range(5):
        off = sid * RPT + i * 125
        pltpu.sync_copy(acc.at[pl.ds(off, 125)], out_hbm.at[cid, pl.ds(off, 125)])


def _edge_phase(ei, q, kv, e):
    mesh = plsc.VectorSubcoreMesh(core_axis_name="c", subcore_axis_name="s")
    fn = functools.partial(
        pl.kernel,
        out_type=jax.ShapeDtypeStruct((NC, N, ACCW), jnp.float32),
        mesh=mesh,
        scratch_types=[
            pltpu.VMEM((2, CH), jnp.int32),
            pltpu.VMEM((2, CH), jnp.int32),
            pltpu.VMEM((CH, HC), jnp.float32),
            pltpu.VMEM((CH, HC), jnp.float32),
            pltpu.VMEM((CH, KV), jnp.float32),
            pltpu.VMEM((CH, KV), jnp.float32),
            pltpu.VMEM((CH, HC), jnp.float32),
            pltpu.VMEM((CH, HC), jnp.float32),
            pltpu.VMEM((TAIL,), jnp.int32),
            pltpu.VMEM((TAIL,), jnp.int32),
            pltpu.VMEM((CH, ACCW), jnp.float32),
            pltpu.VMEM_SHARED((N, ACCW), jnp.float32),
            pltpu.SemaphoreType.DMA,
            pltpu.SemaphoreType.DMA,
            pltpu.SemaphoreType.DMA,
        ],
        compiler_params=pltpu.CompilerParams(use_tc_tiling_on_sc=False,
                                             needs_layout_passes=False),
    )(_edge_body)
    return fn(ei, q, kv, e)


def _combine_body(p_ref, x_ref, ws_ref, bs_ref, o_ref):
    p = p_ref[...]
    s = p[0] + p[1]
    msg = s[:, :HC]
    den4 = s[:, HC:HC + H]
    lane_head = lax.broadcasted_iota(jnp.int32, (H, HC), 1) // C
    head_id = lax.broadcasted_iota(jnp.int32, (H, HC), 0)
    expand = (lane_head == head_id).astype(jnp.float32)
    den = jnp.dot(den4, expand, preferred_element_type=jnp.float32)
    o_ref[...] = (msg / (den + 1e-16)
                  + jnp.dot(x_ref[...], ws_ref[...],
                            preferred_element_type=jnp.float32)
                  + bs_ref[...])


def _combine(partials, x, Wskip, bskip):
    bn = 2000
    return pl.pallas_call(
        _combine_body,
        grid=(N // bn,),
        in_specs=[pl.BlockSpec((NC, bn, ACCW), lambda i: (0, i, 0)),
                  pl.BlockSpec((bn, D), lambda i: (i, 0)),
                  pl.BlockSpec((D, HC), lambda i: (0, 0)),
                  pl.BlockSpec((1, HC), lambda i: (0, 0))],
        out_specs=pl.BlockSpec((bn, HC), lambda i: (i, 0)),
        out_shape=jax.ShapeDtypeStruct((N, HC), jnp.float32),
    )(partials, x, Wskip, bskip.reshape(1, HC))


def kernel(x, edge_index, fold_n, layer, edge_attr,
           Wq, bq, Wk, bk, Wv, bv, We, Wskip, bskip):
    ei = edge_index.astype(jnp.int32)
    q, kv = _projections(x, Wq, bq, Wk, bk, Wv, bv)
    e = _eproj(edge_attr, We)
    partials = _edge_phase(ei, q, kv, e)
    return _combine(partials, x, Wskip, bskip)


# speedup vs baseline: 13.9604x; 1.9591x over previous
import functools

import jax
import jax.numpy as jnp
from jax import lax
from jax.experimental import pallas as pl
from jax.experimental.pallas import tpu as pltpu
from jax.experimental.pallas import tpu_sc as plsc

N = 10000
E = 320000
D = 128
H = 4
C = 32
HC = H * C
KV = 2 * HC
ED = 16
ACCW = 144
INV_SQRT_C = 1.0 / (C ** 0.5)

NC = 2
NS = 16
NW = NC * NS
EPW = E // NW
CH = 32
NCHUNK = EPW // CH
TAIL = EPW - NCHUNK * CH
GRP = CH // 16
RPT = N // NS
NBUF = 2


def _proj_body(x_ref, wq_ref, bq_ref, wk_ref, bk_ref, wv_ref, bv_ref,
               q_ref, kv_ref):
    x = x_ref[...]
    q_ref[...] = jnp.dot(x, wq_ref[...], preferred_element_type=jnp.float32) + bq_ref[...]
    kv_ref[:, :HC] = jnp.dot(x, wk_ref[...], preferred_element_type=jnp.float32) + bk_ref[...]
    kv_ref[:, HC:] = jnp.dot(x, wv_ref[...], preferred_element_type=jnp.float32) + bv_ref[...]


def _projections(x, Wq, bq, Wk, bk, Wv, bv):
    bn = 2000
    grid = (N // bn,)
    row_spec = pl.BlockSpec((bn, D), lambda i: (i, 0))
    w_spec = pl.BlockSpec((D, HC), lambda i: (0, 0))
    b_spec = pl.BlockSpec((1, HC), lambda i: (0, 0))
    return pl.pallas_call(
        _proj_body,
        grid=grid,
        in_specs=[row_spec, w_spec, b_spec, w_spec, b_spec, w_spec, b_spec],
        out_specs=[row_spec, pl.BlockSpec((bn, KV), lambda i: (i, 0))],
        out_shape=[jax.ShapeDtypeStruct((N, HC), jnp.float32),
                   jax.ShapeDtypeStruct((N, KV), jnp.float32)],
    )(x, Wq, bq.reshape(1, HC), Wk, bk.reshape(1, HC), Wv, bv.reshape(1, HC))


def _eproj_body(ea_ref, we_ref, e_ref):
    e_ref[...] = jnp.dot(ea_ref[...], we_ref[...],
                         preferred_element_type=jnp.float32)


def _eproj(edge_attr, We):
    bn = 8000
    return pl.pallas_call(
        _eproj_body,
        grid=(E // bn,),
        in_specs=[pl.BlockSpec((bn, ED), lambda i: (i, 0)),
                  pl.BlockSpec((ED, HC), lambda i: (0, 0))],
        out_specs=pl.BlockSpec((bn, HC), lambda i: (i, 0)),
        out_shape=jax.ShapeDtypeStruct((E, HC), jnp.float32),
    )(edge_attr, We)


def _edge_body(ei_hbm, q_hbm, kv_hbm, e_hbm, out_hbm,
               idx0, idx1, qb0, qb1, kvb0, kvb1, eb0, eb1,
               srcv2, dstv2, msg, acc, gsem0, gsem1, sem):
    idxs = (idx0, idx1)
    qbs = (qb0, qb1)
    kvbs = (kvb0, kvb1)
    ebs = (eb0, eb1)
    gsems = (gsem0, gsem1)

    cid = lax.axis_index("c")
    sid = lax.axis_index("s")
    wid = sid * NC + cid
    ebase = wid * EPW

    zero16 = jnp.zeros((16,), jnp.float32)
    lanes = lax.iota(jnp.int32, 16)

    def _mrow(r, carry):
        for cc in range(ACCW // 16):
            msg[r, pl.ds(cc * 16, 16)] = zero16
        return carry

    lax.fori_loop(0, CH, _mrow, 0)
    for i in range(25):
        pltpu.sync_copy(msg.at[pl.ds(0, 25)],
                        acc.at[pl.ds(sid * RPT + i * 25, 25)])
    plsc.subcore_barrier()

    def _fire(b, t):
        base = ebase + t * CH
        pltpu.sync_copy(ei_hbm.at[:, pl.ds(base, CH)], idxs[b])
        pltpu.async_copy(q_hbm.at[idxs[b].at[1]], qbs[b], gsems[b])
        pltpu.async_copy(kv_hbm.at[idxs[b].at[0]], kvbs[b], gsems[b])
        pltpu.async_copy(e_hbm.at[pl.ds(base, CH)], ebs[b], gsems[b])

    def _drain(b, t):
        base = ebase + t * CH
        pltpu.make_async_copy(q_hbm.at[idxs[b].at[1]], qbs[b], gsems[b]).wait()
        pltpu.make_async_copy(kv_hbm.at[idxs[b].at[0]], kvbs[b], gsems[b]).wait()
        pltpu.make_async_copy(e_hbm.at[pl.ds(base, CH)], ebs[b], gsems[b]).wait()

    def _group(g, q_ref, kv_ref, e_ref, nrow):
        def _edge(j, carry):
            exlane = jnp.zeros((16,), jnp.float32)
            for h in range(H):
                b0, b1 = h * C, h * C + 16
                e0 = e_ref[j, pl.ds(b0, 16)]
                e1 = e_ref[j, pl.ds(b1, 16)]
                p = (q_ref[j, pl.ds(b0, 16)] * (kv_ref[j, pl.ds(b0, 16)] + e0)
                     + q_ref[j, pl.ds(b1, 16)] * (kv_ref[j, pl.ds(b1, 16)] + e1))
                ah = jnp.sum(p) * INV_SQRT_C
                exv = jnp.exp(jnp.full((16,), ah, jnp.float32))
                msg[j, pl.ds(b0, 16)] = exv * (kv_ref[j, pl.ds(HC + b0, 16)] + e0)
                msg[j, pl.ds(b1, 16)] = exv * (kv_ref[j, pl.ds(HC + b1, 16)] + e1)
                exlane = jnp.where(lanes == h, exv, exlane)
            msg[j, pl.ds(HC, 16)] = exlane
            return carry

        lax.fori_loop(g * 16, g * 16 + nrow, _edge, 0)

    for b in range(NBUF):
        _fire(b, b)

    def _outer(i, carry):
        for b in range(NBUF):
            t = i * NBUF + b
            _drain(b, t)
            _group(0, qbs[b], kvbs[b], ebs[b], CH)
            pltpu.sync_copy(msg, acc.at[idxs[b].at[1]], add=True)

            @pl.when(t + NBUF < NCHUNK)
            def _():
                _fire(b, t + NBUF)
        return carry

    lax.fori_loop(0, NCHUNK // NBUF, _outer, 0)

    tbase = ebase + NCHUNK * CH
    pltpu.sync_copy(ei_hbm.at[0, pl.ds(tbase, TAIL)], srcv2)
    pltpu.sync_copy(ei_hbm.at[1, pl.ds(tbase, TAIL)], dstv2)
    cps = [pltpu.async_copy(q_hbm.at[dstv2], qb0.at[pl.ds(0, TAIL)], sem),
           pltpu.async_copy(kv_hbm.at[srcv2], kvb0.at[pl.ds(0, TAIL)], sem),
           pltpu.async_copy(e_hbm.at[pl.ds(tbase, TAIL)],
                            eb0.at[pl.ds(0, TAIL)], sem)]
    for cp in cps:
        cp.wait()
    _group(0, qb0, kvb0, eb0, TAIL)
    pltpu.sync_copy(msg.at[pl.ds(0, TAIL)], acc.at[dstv2], add=True)

    plsc.subcore_barrier()
    for i in range(5):
        off = sid * RPT + i * 125
        pltpu.sync_copy(acc.at[pl.ds(off, 125)], out_hbm.at[cid, pl.ds(off, 125)])


def _edge_phase(ei, q, kv, e):
    mesh = plsc.VectorSubcoreMesh(core_axis_name="c", subcore_axis_name="s")
    fn = functools.partial(
        pl.kernel,
        out_type=jax.ShapeDtypeStruct((NC, N, ACCW), jnp.float32),
        mesh=mesh,
        scratch_types=[
            pltpu.VMEM((2, CH), jnp.int32),
            pltpu.VMEM((2, CH), jnp.int32),
            pltpu.VMEM((CH, HC), jnp.float32),
            pltpu.VMEM((CH, HC), jnp.float32),
            pltpu.VMEM((CH, KV), jnp.float32),
            pltpu.VMEM((CH, KV), jnp.float32),
            pltpu.VMEM((CH, HC), jnp.float32),
            pltpu.VMEM((CH, HC), jnp.float32),
            pltpu.VMEM((TAIL,), jnp.int32),
            pltpu.VMEM((TAIL,), jnp.int32),
            pltpu.VMEM((CH, ACCW), jnp.float32),
            pltpu.VMEM_SHARED((N, ACCW), jnp.float32),
            pltpu.SemaphoreType.DMA,
            pltpu.SemaphoreType.DMA,
            pltpu.SemaphoreType.DMA,
        ],
        compiler_params=pltpu.CompilerParams(use_tc_tiling_on_sc=False,
                                             needs_layout_passes=False),
    )(_edge_body)
    return fn(ei, q, kv, e)


def _combine_body(p_ref, x_ref, ws_ref, bs_ref, o_ref):
    p = p_ref[...]
    s = p[0] + p[1]
    msg = s[:, :HC]
    den4 = s[:, HC:HC + H]
    lane_head = lax.broadcasted_iota(jnp.int32, (H, HC), 1) // C
    head_id = lax.broadcasted_iota(jnp.int32, (H, HC), 0)
    expand = (lane_head == head_id).astype(jnp.float32)
    den = jnp.dot(den4, expand, preferred_element_type=jnp.float32)
    o_ref[...] = (msg / (den + 1e-16)
                  + jnp.dot(x_ref[...], ws_ref[...],
                            preferred_element_type=jnp.float32)
                  + bs_ref[...])


def _combine(partials, x, Wskip, bskip):
    bn = 2000
    return pl.pallas_call(
        _combine_body,
        grid=(N // bn,),
        in_specs=[pl.BlockSpec((NC, bn, ACCW), lambda i: (0, i, 0)),
                  pl.BlockSpec((bn, D), lambda i: (i, 0)),
                  pl.BlockSpec((D, HC), lambda i: (0, 0)),
                  pl.BlockSpec((1, HC), lambda i: (0, 0))],
        out_specs=pl.BlockSpec((bn, HC), lambda i: (i, 0)),
        out_shape=jax.ShapeDtypeStruct((N, HC), jnp.float32),
    )(partials, x, Wskip, bskip.reshape(1, HC))


def kernel(x, edge_index, fold_n, layer, edge_attr,
           Wq, bq, Wk, bk, Wv, bv, We, Wskip, bskip):
    ei = edge_index.astype(jnp.int32)
    q, kv = _projections(x, Wq, bq, Wk, bk, Wv, bv)
    e = _eproj(edge_attr, We)
    partials = _edge_phase(ei, q, kv, e)
    return _combine(partials, x, Wskip, bskip)


# speedup vs baseline: 30.0582x; 2.1531x over previous
import functools

import jax
import jax.numpy as jnp
from jax import lax
from jax.experimental import pallas as pl
from jax.experimental.pallas import tpu as pltpu
from jax.experimental.pallas import tpu_sc as plsc

N = 10000
E = 320000
D = 128
H = 4
C = 32
HC = H * C
KV = 2 * HC
ED = 16
ACCW = 144
INV_SQRT_C = 1.0 / (C ** 0.5)

NC = 2
NS = 16
NW = NC * NS
EPW = E // NW
CH = 32
NCHUNK = EPW // CH
TAIL = EPW - NCHUNK * CH
GRP = CH // 16
RPT = N // NS
NBUF = 2


def _proj_body(x_ref, wq_ref, bq_ref, wk_ref, bk_ref, wv_ref, bv_ref,
               q_ref, kv_ref):
    x = x_ref[...]
    q_ref[...] = jnp.dot(x, wq_ref[...], preferred_element_type=jnp.float32) + bq_ref[...]
    kv_ref[:, :HC] = jnp.dot(x, wk_ref[...], preferred_element_type=jnp.float32) + bk_ref[...]
    kv_ref[:, HC:] = jnp.dot(x, wv_ref[...], preferred_element_type=jnp.float32) + bv_ref[...]


def _projections(x, Wq, bq, Wk, bk, Wv, bv):
    bn = 2000
    grid = (N // bn,)
    row_spec = pl.BlockSpec((bn, D), lambda i: (i, 0))
    w_spec = pl.BlockSpec((D, HC), lambda i: (0, 0))
    b_spec = pl.BlockSpec((1, HC), lambda i: (0, 0))
    return pl.pallas_call(
        _proj_body,
        grid=grid,
        in_specs=[row_spec, w_spec, b_spec, w_spec, b_spec, w_spec, b_spec],
        out_specs=[row_spec, pl.BlockSpec((bn, KV), lambda i: (i, 0))],
        out_shape=[jax.ShapeDtypeStruct((N, HC), jnp.float32),
                   jax.ShapeDtypeStruct((N, KV), jnp.float32)],
    )(x, Wq, bq.reshape(1, HC), Wk, bk.reshape(1, HC), Wv, bv.reshape(1, HC))


def _eproj_body(ea_ref, we_ref, e_ref):
    e_ref[...] = jnp.dot(ea_ref[...], we_ref[...],
                         preferred_element_type=jnp.float32)


def _eproj(edge_attr, We):
    bn = 8000
    return pl.pallas_call(
        _eproj_body,
        grid=(E // bn,),
        in_specs=[pl.BlockSpec((bn, ED), lambda i: (i, 0)),
                  pl.BlockSpec((ED, HC), lambda i: (0, 0))],
        out_specs=pl.BlockSpec((bn, HC), lambda i: (i, 0)),
        out_shape=jax.ShapeDtypeStruct((E, HC), jnp.float32),
    )(edge_attr, We)


def _edge_body(ei_hbm, q_hbm, kv_hbm, e_hbm, out_hbm,
               idx0, idx1, qb0, qb1, kvb0, kvb1, eb0, eb1,
               srcv2, dstv2, msg, acc, gsem0, gsem1, sem):
    idxs = (idx0, idx1)
    qbs = (qb0, qb1)
    kvbs = (kvb0, kvb1)
    ebs = (eb0, eb1)
    gsems = (gsem0, gsem1)

    cid = lax.axis_index("c")
    sid = lax.axis_index("s")
    wid = sid * NC + cid
    ebase = wid * EPW

    zero16 = jnp.zeros((16,), jnp.float32)
    lanes = lax.iota(jnp.int32, 16)

    def _mrow(r, carry):
        for cc in range(ACCW // 16):
            msg[r, pl.ds(cc * 16, 16)] = zero16
        return carry

    lax.fori_loop(0, CH, _mrow, 0)
    for i in range(25):
        pltpu.sync_copy(msg.at[pl.ds(0, 25)],
                        acc.at[pl.ds(sid * RPT + i * 25, 25)])
    plsc.subcore_barrier()

    def _fire(b, t):
        base = ebase + t * CH
        pltpu.sync_copy(ei_hbm.at[:, pl.ds(base, CH)], idxs[b])
        pltpu.async_copy(q_hbm.at[idxs[b].at[1]], qbs[b], gsems[b])
        pltpu.async_copy(kv_hbm.at[idxs[b].at[0]], kvbs[b], gsems[b])
        pltpu.async_copy(e_hbm.at[pl.ds(base, CH)], ebs[b], gsems[b])

    def _drain(b, t):
        base = ebase + t * CH
        pltpu.make_async_copy(q_hbm.at[idxs[b].at[1]], qbs[b], gsems[b]).wait()
        pltpu.make_async_copy(kv_hbm.at[idxs[b].at[0]], kvbs[b], gsems[b]).wait()
        pltpu.make_async_copy(e_hbm.at[pl.ds(base, CH)], ebs[b], gsems[b]).wait()

    def _group(g, q_ref, kv_ref, e_ref, nrow):
        @plsc.parallel_loop(g * 16, g * 16 + nrow, unroll=4)
        def _edge(j):
            exlane = jnp.zeros((16,), jnp.float32)
            for h in range(H):
                b0, b1 = h * C, h * C + 16
                e0 = e_ref[j, pl.ds(b0, 16)]
                e1 = e_ref[j, pl.ds(b1, 16)]
                p = (q_ref[j, pl.ds(b0, 16)] * (kv_ref[j, pl.ds(b0, 16)] + e0)
                     + q_ref[j, pl.ds(b1, 16)] * (kv_ref[j, pl.ds(b1, 16)] + e1))
                ah = jnp.sum(p) * INV_SQRT_C
                exv = jnp.exp(jnp.full((16,), ah, jnp.float32))
                msg[j, pl.ds(b0, 16)] = exv * (kv_ref[j, pl.ds(HC + b0, 16)] + e0)
                msg[j, pl.ds(b1, 16)] = exv * (kv_ref[j, pl.ds(HC + b1, 16)] + e1)
                exlane = jnp.where(lanes == h, exv, exlane)
            msg[j, pl.ds(HC, 16)] = exlane

    for b in range(NBUF):
        _fire(b, b)

    def _outer(i, carry):
        for b in range(NBUF):
            t = i * NBUF + b
            _drain(b, t)
            _group(0, qbs[b], kvbs[b], ebs[b], CH)
            pltpu.sync_copy(msg, acc.at[idxs[b].at[1]], add=True)

            @pl.when(t + NBUF < NCHUNK)
            def _():
                _fire(b, t + NBUF)
        return carry

    lax.fori_loop(0, NCHUNK // NBUF, _outer, 0)

    tbase = ebase + NCHUNK * CH
    pltpu.sync_copy(ei_hbm.at[0, pl.ds(tbase, TAIL)], srcv2)
    pltpu.sync_copy(ei_hbm.at[1, pl.ds(tbase, TAIL)], dstv2)
    cps = [pltpu.async_copy(q_hbm.at[dstv2], qb0.at[pl.ds(0, TAIL)], sem),
           pltpu.async_copy(kv_hbm.at[srcv2], kvb0.at[pl.ds(0, TAIL)], sem),
           pltpu.async_copy(e_hbm.at[pl.ds(tbase, TAIL)],
                            eb0.at[pl.ds(0, TAIL)], sem)]
    for cp in cps:
        cp.wait()
    _group(0, qb0, kvb0, eb0, TAIL)
    pltpu.sync_copy(msg.at[pl.ds(0, TAIL)], acc.at[dstv2], add=True)

    plsc.subcore_barrier()
    for i in range(5):
        off = sid * RPT + i * 125
        pltpu.sync_copy(acc.at[pl.ds(off, 125)], out_hbm.at[cid, pl.ds(off, 125)])


def _edge_phase(ei, q, kv, e):
    mesh = plsc.VectorSubcoreMesh(core_axis_name="c", subcore_axis_name="s")
    fn = functools.partial(
        pl.kernel,
        out_type=jax.ShapeDtypeStruct((NC, N, ACCW), jnp.float32),
        mesh=mesh,
        scratch_types=[
            pltpu.VMEM((2, CH), jnp.int32),
            pltpu.VMEM((2, CH), jnp.int32),
            pltpu.VMEM((CH, HC), jnp.float32),
            pltpu.VMEM((CH, HC), jnp.float32),
            pltpu.VMEM((CH, KV), jnp.float32),
            pltpu.VMEM((CH, KV), jnp.float32),
            pltpu.VMEM((CH, HC), jnp.float32),
            pltpu.VMEM((CH, HC), jnp.float32),
            pltpu.VMEM((TAIL,), jnp.int32),
            pltpu.VMEM((TAIL,), jnp.int32),
            pltpu.VMEM((CH, ACCW), jnp.float32),
            pltpu.VMEM_SHARED((N, ACCW), jnp.float32),
            pltpu.SemaphoreType.DMA,
            pltpu.SemaphoreType.DMA,
            pltpu.SemaphoreType.DMA,
        ],
        compiler_params=pltpu.CompilerParams(use_tc_tiling_on_sc=False,
                                             needs_layout_passes=False),
    )(_edge_body)
    return fn(ei, q, kv, e)


def _combine_body(p_ref, x_ref, ws_ref, bs_ref, o_ref):
    p = p_ref[...]
    s = p[0] + p[1]
    msg = s[:, :HC]
    den4 = s[:, HC:HC + H]
    lane_head = lax.broadcasted_iota(jnp.int32, (H, HC), 1) // C
    head_id = lax.broadcasted_iota(jnp.int32, (H, HC), 0)
    expand = (lane_head == head_id).astype(jnp.float32)
    den = jnp.dot(den4, expand, preferred_element_type=jnp.float32)
    o_ref[...] = (msg / (den + 1e-16)
                  + jnp.dot(x_ref[...], ws_ref[...],
                            preferred_element_type=jnp.float32)
                  + bs_ref[...])


def _combine(partials, x, Wskip, bskip):
    bn = 2000
    return pl.pallas_call(
        _combine_body,
        grid=(N // bn,),
        in_specs=[pl.BlockSpec((NC, bn, ACCW), lambda i: (0, i, 0)),
                  pl.BlockSpec((bn, D), lambda i: (i, 0)),
                  pl.BlockSpec((D, HC), lambda i: (0, 0)),
                  pl.BlockSpec((1, HC), lambda i: (0, 0))],
        out_specs=pl.BlockSpec((bn, HC), lambda i: (i, 0)),
        out_shape=jax.ShapeDtypeStruct((N, HC), jnp.float32),
    )(partials, x, Wskip, bskip.reshape(1, HC))


def kernel(x, edge_index, fold_n, layer, edge_attr,
           Wq, bq, Wk, bk, Wv, bv, We, Wskip, bskip):
    ei = edge_index.astype(jnp.int32)
    q, kv = _projections(x, Wq, bq, Wk, bk, Wv, bv)
    e = _eproj(edge_attr, We)
    partials = _edge_phase(ei, q, kv, e)
    return _combine(partials, x, Wskip, bskip)
